# baseline (speedup 1.0000x reference)
import jax
import jax.numpy as jnp
from jax import lax
from jax.experimental import pallas as pl
from jax.experimental.pallas import tpu as pltpu
from jax.experimental.pallas import tpu_sc as plsc

_EMBED = 64
_MAXLEN = 200
_BATCH = 4096
_VOCAB = 1000000

_NC = 2
_NS = 16
_NW = _NC * _NS

_ROWS_W = _BATCH // _NW
_G0 = 104
_G1 = _MAXLEN - _G0
_LANES = 16
_NBUF = 4
_TOTAL = _BATCH * _MAXLEN

_PBLK = 256
_PFULL = _VOCAB // _PBLK
_PREM = _VOCAB - _PFULL * _PBLK


def _pack_body(table_hbm, out_hbm, b64_0, b64_1, b128_0, b128_1,
               rsem0, rsem1, wsem0, wsem1):
    wid = lax.axis_index("s") * _NC + lax.axis_index("c")
    b64 = (b64_0, b64_1)
    b128 = (b128_0, b128_1)
    rsems = (rsem0, rsem1)
    wsems = (wsem0, wsem1)

    def _rd(k, s):
        pltpu.async_copy(table_hbm.at[pl.ds(k * _PBLK, _PBLK)], b64[s], rsems[s])

    def _rd_wait(s):
        pltpu.make_async_copy(
            table_hbm.at[pl.ds(0, _PBLK)], b64[s], rsems[s]
        ).wait()

    def _vcopy(s, nrows):
        @plsc.parallel_loop(0, nrows, unroll=2)
        def _(r):
            for c in range(0, _EMBED, _LANES):
                b128[s][r, pl.ds(c, _LANES)] = b64[s][r, pl.ds(c, _LANES)]

    def _wr(k, s):
        pltpu.async_copy(b128[s], out_hbm.at[pl.ds(k * _PBLK, _PBLK)], wsems[s])

    def _wr_wait(s):
        pltpu.make_async_copy(
            b128[s], out_hbm.at[pl.ds(0, _PBLK)], wsems[s]
        ).wait()

    _rd(wid, 0)

    @pl.loop(wid, _PFULL, step=2 * _NW)
    def _blk(k):
        @pl.when(k + _NW < _PFULL)
        def _():
            _rd(k + _NW, 1)

        _rd_wait(0)

        @pl.when(k > wid)
        def _():
            _wr_wait(0)

        _vcopy(0, _PBLK)
        _wr(k, 0)

        @pl.when(k + 2 * _NW < _PFULL)
        def _():
            _rd(k + 2 * _NW, 0)

        @pl.when(k + _NW < _PFULL)
        def _():
            _rd_wait(1)

            @pl.when(k > wid)
            def _():
                _wr_wait(1)

            _vcopy(1, _PBLK)
            _wr(k + _NW, 1)

    _wr_wait(0)
    _wr_wait(1)

    @pl.when(wid == 0)
    def _():
        pltpu.async_copy(
            table_hbm.at[pl.ds(_PFULL * _PBLK, _PREM)],
            b64_0.at[pl.ds(0, _PREM)],
            rsem0,
        ).wait()
        _vcopy(0, _PREM)
        pltpu.async_copy(
            b128_0.at[pl.ds(0, _PREM)],
            out_hbm.at[pl.ds(_PFULL * _PBLK, _PREM)],
            wsem0,
        ).wait()


def _emb_body(table_hbm, idx_hbm, pos_hbm, out_hbm, pos_v, *rest):
    ibufs = rest[0:_NBUF]
    abufs = rest[_NBUF:2 * _NBUF]
    isems = rest[2 * _NBUF:3 * _NBUF]
    gsems = rest[3 * _NBUF:4 * _NBUF]
    wsems = rest[4 * _NBUF:5 * _NBUF]

    wid = lax.axis_index("s") * _NC + lax.axis_index("c")
    row0 = wid * _ROWS_W

    pltpu.sync_copy(pos_hbm, pos_v)

    def _idx_dma(j, s):
        pltpu.async_copy(idx_hbm.at[row0 + j], ibufs[s], isems[s])

    def _gathers(j, s):
        pltpu.async_copy(
            table_hbm.at[ibufs[s].at[pl.ds(0, _G0)]],
            abufs[s].at[pl.ds(0, _G0)],
            gsems[s],
        )
        pltpu.async_copy(
            table_hbm.at[ibufs[s].at[pl.ds(_G0, _G1)]],
            abufs[s].at[pl.ds(_G0, _G1)],
            gsems[s],
        )

    def _wait_gathers(s):
        pltpu.make_async_copy(
            table_hbm.at[ibufs[s].at[pl.ds(0, _G0)]],
            abufs[s].at[pl.ds(0, _G0)],
            gsems[s],
        ).wait()
        pltpu.make_async_copy(
            table_hbm.at[ibufs[s].at[pl.ds(_G0, _G1)]],
            abufs[s].at[pl.ds(_G0, _G1)],
            gsems[s],
        ).wait()

    def _wait_idx(s):
        pltpu.make_async_copy(idx_hbm.at[row0], ibufs[s], isems[s]).wait()

    def _wb(j, s):
        pltpu.async_copy(
            abufs[s],
            out_hbm.at[pl.ds((row0 + j) * _MAXLEN, _MAXLEN), pl.ds(0, _EMBED)],
            wsems[s],
        )

    def _wait_wb(s):
        pltpu.make_async_copy(
            abufs[s],
            out_hbm.at[pl.ds(row0 * _MAXLEN, _MAXLEN), pl.ds(0, _EMBED)],
            wsems[s],
        ).wait()

    for s in range(_NBUF):
        _idx_dma(s, s)
    for jg in range(2):
        _wait_idx(jg)
        _gathers(jg, jg)

    @pl.loop(0, _ROWS_W, step=_NBUF)
    def _steps(j0):
        for b in range(_NBUF):
            j = j0 + b
            sg = (b + 2) % _NBUF

            @pl.when(j < _ROWS_W - 2)
            def _():
                _wait_idx(sg)

                @pl.when(j >= 2)
                def _():
                    _wait_wb(sg)

                _gathers(j + 2, sg)

            _wait_gathers(b)

            @pl.when(j < _ROWS_W - _NBUF)
            def _():
                _idx_dma(j + _NBUF, b)

            @plsc.parallel_loop(0, _MAXLEN, unroll=2)
            def _row(r):
                for c in range(0, _EMBED, _LANES):
                    abufs[b][r, pl.ds(c, _LANES)] = (
                        abufs[b][r, pl.ds(c, _LANES)] + pos_v[r, pl.ds(c, _LANES)]
                    )

            _wb(j, b)

    for s in range(_NBUF):
        _wait_wb(s)


def kernel(inputs, token_table, pos_emb):
    mesh = plsc.VectorSubcoreMesh(core_axis_name="c", subcore_axis_name="s")

    pack = pl.kernel(
        _pack_body,
        out_type=jax.ShapeDtypeStruct((_VOCAB, 128), jnp.float32),
        mesh=mesh,
        compiler_params=pltpu.CompilerParams(use_tc_tiling_on_sc=True),
        scratch_types=[
            pltpu.VMEM((_PBLK, _EMBED), jnp.float32),
            pltpu.VMEM((_PBLK, _EMBED), jnp.float32),
            pltpu.VMEM((_PBLK, 128), jnp.float32),
            pltpu.VMEM((_PBLK, 128), jnp.float32),
            pltpu.SemaphoreType.DMA,
            pltpu.SemaphoreType.DMA,
            pltpu.SemaphoreType.DMA,
            pltpu.SemaphoreType.DMA,
        ],
    )
    table2 = pack(token_table).reshape(2 * _VOCAB, _EMBED)

    idx = (inputs * 2).astype(jnp.int32)
    scratch = (
        [pltpu.VMEM((_MAXLEN, _EMBED), jnp.float32)]
        + [pltpu.VMEM((_MAXLEN,), jnp.int32) for _ in range(_NBUF)]
        + [pltpu.VMEM((_MAXLEN, _EMBED), jnp.float32) for _ in range(_NBUF)]
        + [pltpu.SemaphoreType.DMA for _ in range(3 * _NBUF)]
    )
    k = pl.kernel(
        _emb_body,
        out_type=jax.ShapeDtypeStruct((_TOTAL, 128), jnp.float32),
        mesh=mesh,
        compiler_params=pltpu.CompilerParams(use_tc_tiling_on_sc=False),
        scratch_types=scratch,
    )
    out = k(table2, idx, pos_emb)
    return out[:, :_EMBED].reshape(_BATCH, _MAXLEN, _EMBED)


# speedup vs baseline: 1.1002x; 1.1002x over previous
import jax
import jax.numpy as jnp
from jax import lax
from jax.experimental import pallas as pl
from jax.experimental.pallas import tpu as pltpu
from jax.experimental.pallas import tpu_sc as plsc

_EMBED = 64
_MAXLEN = 200
_BATCH = 4096
_VOCAB = 1000000

_NC = 2
_NS = 16
_NW = _NC * _NS

_ROWS_W = _BATCH // _NW
_G0 = 104
_G1 = _MAXLEN - _G0
_LANES = 16
_NBUF = 4
_TOTAL = _BATCH * _MAXLEN


def _emb_body(table_hbm, idx_hbm, pos_hbm, out_hbm, pos_v, *rest):
    ibufs = rest[0:_NBUF]
    abufs = rest[_NBUF:2 * _NBUF]
    isems = rest[2 * _NBUF:3 * _NBUF]
    gsems = rest[3 * _NBUF:4 * _NBUF]
    wsems = rest[4 * _NBUF:5 * _NBUF]

    wid = lax.axis_index("s") * _NC + lax.axis_index("c")
    row0 = wid * _ROWS_W

    pltpu.sync_copy(pos_hbm, pos_v)

    def _idx_dma(j, s):
        pltpu.async_copy(idx_hbm.at[row0 + j], ibufs[s], isems[s])

    def _gathers(j, s):
        pltpu.async_copy(
            table_hbm.at[ibufs[s].at[pl.ds(0, _G0)]],
            abufs[s].at[pl.ds(0, _G0)],
            gsems[s],
        )
        pltpu.async_copy(
            table_hbm.at[ibufs[s].at[pl.ds(_G0, _G1)]],
            abufs[s].at[pl.ds(_G0, _G1)],
            gsems[s],
        )

    def _wait_gathers(s):
        pltpu.make_async_copy(
            table_hbm.at[ibufs[s].at[pl.ds(0, _G0)]],
            abufs[s].at[pl.ds(0, _G0)],
            gsems[s],
        ).wait()
        pltpu.make_async_copy(
            table_hbm.at[ibufs[s].at[pl.ds(_G0, _G1)]],
            abufs[s].at[pl.ds(_G0, _G1)],
            gsems[s],
        ).wait()

    def _wait_idx(s):
        pltpu.make_async_copy(idx_hbm.at[row0], ibufs[s], isems[s]).wait()

    def _wb(j, s):
        pltpu.async_copy(
            abufs[s],
            out_hbm.at[pl.ds((row0 + j) * _MAXLEN, _MAXLEN), pl.ds(0, _EMBED)],
            wsems[s],
        )

    def _wait_wb(s):
        pltpu.make_async_copy(
            abufs[s],
            out_hbm.at[pl.ds(row0 * _MAXLEN, _MAXLEN), pl.ds(0, _EMBED)],
            wsems[s],
        ).wait()

    for s in range(_NBUF):
        _idx_dma(s, s)
    for jg in range(2):
        _wait_idx(jg)
        _gathers(jg, jg)

    @pl.loop(0, _ROWS_W, step=_NBUF)
    def _steps(j0):
        for b in range(_NBUF):
            j = j0 + b
            sg = (b + 2) % _NBUF

            @pl.when(j < _ROWS_W - 2)
            def _():
                _wait_idx(sg)

                @pl.when(j >= 2)
                def _():
                    _wait_wb(sg)

                _gathers(j + 2, sg)

            _wait_gathers(b)

            @pl.when(j < _ROWS_W - _NBUF)
            def _():
                _idx_dma(j + _NBUF, b)

            @plsc.parallel_loop(0, _MAXLEN, unroll=2)
            def _row(r):
                for c in range(0, _EMBED, _LANES):
                    abufs[b][r, pl.ds(c, _LANES)] = (
                        abufs[b][r, pl.ds(c, _LANES)] + pos_v[r, pl.ds(c, _LANES)]
                    )

            _wb(j, b)

    for s in range(_NBUF):
        _wait_wb(s)


def kernel(inputs, token_table, pos_emb):
    mesh = plsc.VectorSubcoreMesh(core_axis_name="c", subcore_axis_name="s")
    idx = inputs.astype(jnp.int32)
    scratch = (
        [pltpu.VMEM((_MAXLEN, _EMBED), jnp.float32)]
        + [pltpu.VMEM((_MAXLEN,), jnp.int32) for _ in range(_NBUF)]
        + [pltpu.VMEM((_MAXLEN, _EMBED), jnp.float32) for _ in range(_NBUF)]
        + [pltpu.SemaphoreType.DMA for _ in range(3 * _NBUF)]
    )
    k = pl.kernel(
        _emb_body,
        out_type=jax.ShapeDtypeStruct((_TOTAL, 128), jnp.float32),
        mesh=mesh,
        compiler_params=pltpu.CompilerParams(use_tc_tiling_on_sc=False),
        scratch_types=scratch,
    )
    out = k(token_table, idx, pos_emb)
    return out[:, :_EMBED].reshape(_BATCH, _MAXLEN, _EMBED)
